# baseline (speedup 1.0000x reference)
import dataclasses
import functools

import jax
import jax.numpy as jnp
from jax import lax
from jax.experimental import pallas as pl
from jax.experimental.pallas import tpu as pltpu
from jax.experimental.pallas import tpu_sc as plsc

N = 10000
E = 10000
NI = 320000
D = 128
CW = 16
NC = 2
NS = 16
CHUNK = 80
PER_CORE = NI // NC
NCHUNK = PER_CORE // CHUNK
NFULL = (NCHUNK // NS) - (NCHUNK // NS) % 2
ZROWS = 80
NZBLK = N // ZROWS
BLK = 1000


def _sc_segment_sum(table, gidx, sidx):
    mesh = plsc.VectorSubcoreMesh(core_axis_name="c", subcore_axis_name="s")

    @functools.partial(
        pl.kernel,
        out_type=[
            jax.ShapeDtypeStruct((NC, N, D), jnp.float32),
            jax.ShapeDtypeStruct((NC, NS, N), jnp.float32),
        ],
        mesh=mesh,
        scratch_types=[
            pltpu.VMEM((2, CHUNK), jnp.int32),
            pltpu.VMEM((2, CHUNK), jnp.int32),
            pltpu.VMEM((CHUNK, D), jnp.float32),
            pltpu.VMEM((CHUNK, D), jnp.float32),
            pltpu.VMEM((N,), jnp.float32),
            pltpu.VMEM_SHARED((N, D), jnp.float32),
            pltpu.SemaphoreType.DMA,
            pltpu.SemaphoreType.DMA,
            pltpu.SemaphoreType.DMA,
            pltpu.SemaphoreType.DMA,
        ],
        compiler_params=dataclasses.replace(
            pltpu.CompilerParams(), needs_layout_passes=False),
    )
    def seg_kernel(table_hbm, gidx_hbm, sidx_hbm, zf_hbm,
                   out_hbm, cnt_hbm,
                   gi_v, si_v, b0_v, b1_v, hist_v, acc_sh,
                   gsem0, gsem1, isem0, isem1):
        c = lax.axis_index("c")
        s = lax.axis_index("s")
        bufs = (b0_v, b1_v)
        gsems = (gsem0, gsem1)
        isems = (isem0, isem1)

        @pl.loop(0, N // 16)
        def _(i):
            hist_v[pl.ds(i * 16, 16)] = jnp.zeros((16,), jnp.float32)

        pltpu.sync_copy(zf_hbm, b0_v)

        @pl.loop(s, NZBLK, step=NS)
        def _(t):
            pltpu.sync_copy(b0_v, acc_sh.at[pl.ds(t * ZROWS, ZROWS)])

        plsc.subcore_barrier()

        def chunk_base(i):
            return c * PER_CORE + (s + i * NS) * CHUNK

        def start_idx_fetch(i, slot):
            base = chunk_base(i)
            pltpu.async_copy(gidx_hbm.at[pl.ds(base, CHUNK)],
                             gi_v.at[slot], isems[slot])
            pltpu.async_copy(sidx_hbm.at[pl.ds(base, CHUNK)],
                             si_v.at[slot], isems[slot])

        def wait_idx(slot):
            pltpu.make_async_copy(gidx_hbm.at[pl.ds(0, CHUNK)],
                                  gi_v.at[slot], isems[slot]).wait()
            pltpu.make_async_copy(gidx_hbm.at[pl.ds(0, CHUNK)],
                                  si_v.at[slot], isems[slot]).wait()

        def start_gather(slot):
            pltpu.async_copy(table_hbm.at[gi_v.at[slot]],
                             bufs[slot], gsems[slot])

        def wait_gather(slot):
            pltpu.make_async_copy(table_hbm.at[pl.ds(0, CHUNK)],
                                  bufs[slot], gsems[slot]).wait()

        def cnt_scatter(slot):
            for j in range(CHUNK // 16):
                idx = si_v[slot, pl.ds(j * 16, 16)]
                plsc.addupdate_scatter(hist_v, [idx],
                                       jnp.ones((16,), jnp.float32))

        def scatter_add(slot):
            pltpu.sync_copy(bufs[slot], acc_sh.at[si_v.at[slot]], add=True)

        base0 = chunk_base(0)
        pltpu.sync_copy(gidx_hbm.at[pl.ds(base0, CHUNK)], gi_v.at[0])
        pltpu.sync_copy(sidx_hbm.at[pl.ds(base0, CHUNK)], si_v.at[0])
        start_idx_fetch(1, 1)
        start_gather(0)

        @pl.loop(0, NFULL // 2 - 1)
        def _(p):
            i0 = 2 * p
            wait_idx(1)
            start_gather(1)
            cnt_scatter(0)
            wait_gather(0)
            scatter_add(0)
            start_idx_fetch(i0 + 2, 0)
            wait_idx(0)
            start_gather(0)
            cnt_scatter(1)
            wait_gather(1)
            scatter_add(1)
            start_idx_fetch(i0 + 3, 1)

        wait_idx(1)
        start_gather(1)
        cnt_scatter(0)
        wait_gather(0)
        scatter_add(0)
        cnt_scatter(1)
        wait_gather(1)
        scatter_add(1)

        for i in range(NFULL, NCHUNK // NS):
            base = chunk_base(i)
            pltpu.sync_copy(gidx_hbm.at[pl.ds(base, CHUNK)], gi_v.at[0])
            pltpu.sync_copy(sidx_hbm.at[pl.ds(base, CHUNK)], si_v.at[0])
            pltpu.async_copy(table_hbm.at[gi_v.at[0]], b0_v, gsem0).wait()
            cnt_scatter(0)
            scatter_add(0)

        plsc.subcore_barrier()

        pltpu.sync_copy(hist_v, cnt_hbm.at[c, s])

        @pl.loop(s, NZBLK, step=NS)
        def _(t):
            row0 = t * ZROWS
            pltpu.sync_copy(acc_sh.at[pl.ds(row0, ZROWS)],
                            b1_v.at[pl.ds(0, ZROWS)])
            pltpu.sync_copy(b1_v.at[pl.ds(0, ZROWS)],
                            out_hbm.at[c, pl.ds(row0, ZROWS)])

    zf = jnp.zeros((ZROWS, D), jnp.float32)
    return seg_kernel(table, gidx, sidx, zf)


def _tc_front(x, W1, b1, Wu, bu):
    def body(x_ref, w1_ref, b1_ref, wu_ref, bu_ref, m_ref, u_ref):
        xb = x_ref[...]
        m_ref[...] = jnp.maximum(
            jnp.dot(xb, w1_ref[...], preferred_element_type=jnp.float32)
            + b1_ref[...], 0.0)
        u_ref[...] = (
            jnp.dot(xb, wu_ref[...], preferred_element_type=jnp.float32)
            + bu_ref[...])

    return pl.pallas_call(
        body,
        grid=(N // BLK,),
        in_specs=[
            pl.BlockSpec((BLK, D), lambda i: (i, 0)),
            pl.BlockSpec((D, D), lambda i: (0, 0)),
            pl.BlockSpec((1, D), lambda i: (0, 0)),
            pl.BlockSpec((D, D), lambda i: (0, 0)),
            pl.BlockSpec((1, D), lambda i: (0, 0)),
        ],
        out_specs=[
            pl.BlockSpec((BLK, D), lambda i: (i, 0)),
            pl.BlockSpec((BLK, D), lambda i: (i, 0)),
        ],
        out_shape=[
            jax.ShapeDtypeStruct((N, D), jnp.float32),
            jax.ShapeDtypeStruct((N, D), jnp.float32),
        ],
    )(x, W1, b1.reshape(1, D), Wu, bu.reshape(1, D))


NPAD = 10240
CBLK = 1280


def _tc_cnt(cnts):
    padded = jnp.pad(cnts.reshape(NC * NS, N), ((0, 0), (0, NPAD - N)))

    def body(c_ref, o_ref):
        ones = jnp.ones((NC * NS, 1), jnp.float32)
        o_ref[...] = lax.dot_general(
            c_ref[...], ones, (((0,), (0,)), ((), ())),
            preferred_element_type=jnp.float32)

    out = pl.pallas_call(
        body,
        grid=(NPAD // CBLK,),
        in_specs=[pl.BlockSpec((NC * NS, CBLK), lambda i: (0, i))],
        out_specs=pl.BlockSpec((CBLK, 1), lambda i: (i, 0)),
        out_shape=jax.ShapeDtypeStruct((NPAD, 1), jnp.float32),
    )(padded)
    return out[:N]


def _tc_mid(parts, cnt_col):
    def body(p_ref, c_ref, o_ref):
        o_ref[...] = (p_ref[0] + p_ref[1]) / jnp.maximum(c_ref[...], 1.0)

    return pl.pallas_call(
        body,
        grid=(E // BLK,),
        in_specs=[
            pl.BlockSpec((NC, BLK, D), lambda i: (0, i, 0)),
            pl.BlockSpec((BLK, 1), lambda i: (i, 0)),
        ],
        out_specs=pl.BlockSpec((BLK, D), lambda i: (i, 0)),
        out_shape=jax.ShapeDtypeStruct((E, D), jnp.float32),
    )(parts, cnt_col)


def _tc_back(u, parts, cnt_col):
    def body(u_ref, p_ref, c_ref, o_ref):
        mi = (p_ref[0] + p_ref[1]) / jnp.maximum(c_ref[...], 1.0)
        h = jnp.maximum(u_ref[...] + mi, 0.0)
        mx = jnp.max(h, axis=1, keepdims=True)
        lse = jnp.log(jnp.sum(jnp.exp(h - mx), axis=1, keepdims=True))
        o_ref[...] = h - mx - lse

    return pl.pallas_call(
        body,
        grid=(N // BLK,),
        in_specs=[
            pl.BlockSpec((BLK, D), lambda i: (i, 0)),
            pl.BlockSpec((NC, BLK, D), lambda i: (0, i, 0)),
            pl.BlockSpec((BLK, 1), lambda i: (i, 0)),
        ],
        out_specs=pl.BlockSpec((BLK, D), lambda i: (i, 0)),
        out_shape=jax.ShapeDtypeStruct((N, D), jnp.float32),
    )(u, parts, cnt_col)


def kernel(x, edge_index, W1, b1, Wu, bu):
    v_idx = edge_index[0]
    e_idx = edge_index[1]
    m, u = _tc_front(x, W1, b1, Wu, bu)
    e_parts, e_cnts = _sc_segment_sum(m, v_idx, e_idx)
    e_feat = _tc_mid(e_parts, _tc_cnt(e_cnts))
    v_parts, v_cnts = _sc_segment_sum(e_feat, e_idx, v_idx)
    return _tc_back(u, v_parts, _tc_cnt(v_cnts))


# speedup vs baseline: 1.1383x; 1.1383x over previous
import dataclasses
import functools

import jax
import jax.numpy as jnp
from jax import lax
from jax.experimental import pallas as pl
from jax.experimental.pallas import tpu as pltpu
from jax.experimental.pallas import tpu_sc as plsc

N = 10000
E = 10000
NI = 320000
D = 128
CW = 16
NC = 2
NS = 16
CHUNK = 128
PER_TILE = NI // (NC * NS)
NFULL = (PER_TILE // CHUNK) - (PER_TILE // CHUNK) % 2
TAIL = PER_TILE - NFULL * CHUNK
ZROWS = 80
NZBLK = N // ZROWS
BLK = 1000


def _sc_segment_sum(table, gidx, sidx):
    mesh = plsc.VectorSubcoreMesh(core_axis_name="c", subcore_axis_name="s")

    @functools.partial(
        pl.kernel,
        out_type=[
            jax.ShapeDtypeStruct((NC, N, D), jnp.float32),
            jax.ShapeDtypeStruct((NC, NS, N), jnp.float32),
        ],
        mesh=mesh,
        scratch_types=[
            pltpu.VMEM((2, CHUNK), jnp.int32),
            pltpu.VMEM((2, CHUNK), jnp.int32),
            pltpu.VMEM((1, TAIL), jnp.int32),
            pltpu.VMEM((1, TAIL), jnp.int32),
            pltpu.VMEM((CHUNK, D), jnp.float32),
            pltpu.VMEM((CHUNK, D), jnp.float32),
            pltpu.VMEM((N,), jnp.float32),
            pltpu.VMEM_SHARED((N, D), jnp.float32),
            pltpu.SemaphoreType.DMA,
            pltpu.SemaphoreType.DMA,
            pltpu.SemaphoreType.DMA,
            pltpu.SemaphoreType.DMA,
        ],
        compiler_params=dataclasses.replace(
            pltpu.CompilerParams(), needs_layout_passes=False),
    )
    def seg_kernel(table_hbm, gidx_hbm, sidx_hbm, zf_hbm,
                   out_hbm, cnt_hbm,
                   gi_v, si_v, gi_t, si_t, b0_v, b1_v, hist_v, acc_sh,
                   gsem0, gsem1, isem0, isem1):
        c = lax.axis_index("c")
        s = lax.axis_index("s")
        tile_base = (c * NS + s) * PER_TILE
        bufs = (b0_v, b1_v)
        gsems = (gsem0, gsem1)
        isems = (isem0, isem1)

        @pl.loop(0, N // 16)
        def _(i):
            hist_v[pl.ds(i * 16, 16)] = jnp.zeros((16,), jnp.float32)

        pltpu.sync_copy(zf_hbm, b0_v.at[pl.ds(0, ZROWS)])

        @pl.loop(s, NZBLK, step=NS)
        def _(t):
            pltpu.sync_copy(b0_v.at[pl.ds(0, ZROWS)],
                            acc_sh.at[pl.ds(t * ZROWS, ZROWS)])

        plsc.subcore_barrier()

        def chunk_base(i):
            return tile_base + i * CHUNK

        def start_idx_fetch(i, slot):
            base = chunk_base(i)
            pltpu.async_copy(gidx_hbm.at[pl.ds(base, CHUNK)],
                             gi_v.at[slot], isems[slot])
            pltpu.async_copy(sidx_hbm.at[pl.ds(base, CHUNK)],
                             si_v.at[slot], isems[slot])

        def wait_idx(slot):
            pltpu.make_async_copy(gidx_hbm.at[pl.ds(0, CHUNK)],
                                  gi_v.at[slot], isems[slot]).wait()
            pltpu.make_async_copy(gidx_hbm.at[pl.ds(0, CHUNK)],
                                  si_v.at[slot], isems[slot]).wait()

        def start_gather(slot):
            pltpu.async_copy(table_hbm.at[gi_v.at[slot]],
                             bufs[slot], gsems[slot])

        def wait_gather(slot):
            pltpu.make_async_copy(table_hbm.at[pl.ds(0, CHUNK)],
                                  bufs[slot], gsems[slot]).wait()

        def cnt_scatter(slot):
            for j in range(CHUNK // 16):
                idx = si_v[slot, pl.ds(j * 16, 16)]
                plsc.addupdate_scatter(hist_v, [idx],
                                       jnp.ones((16,), jnp.float32))

        def scatter_add(slot):
            pltpu.sync_copy(bufs[slot], acc_sh.at[si_v.at[slot]], add=True)

        base0 = chunk_base(0)
        pltpu.sync_copy(gidx_hbm.at[pl.ds(base0, CHUNK)], gi_v.at[0])
        pltpu.sync_copy(sidx_hbm.at[pl.ds(base0, CHUNK)], si_v.at[0])
        start_idx_fetch(1, 1)
        start_gather(0)

        @pl.loop(0, NFULL // 2 - 1)
        def _(p):
            i0 = 2 * p
            wait_idx(1)
            start_gather(1)
            cnt_scatter(0)
            wait_gather(0)
            scatter_add(0)
            start_idx_fetch(i0 + 2, 0)
            wait_idx(0)
            start_gather(0)
            cnt_scatter(1)
            wait_gather(1)
            scatter_add(1)
            start_idx_fetch(i0 + 3, 1)

        wait_idx(1)
        start_gather(1)
        cnt_scatter(0)
        wait_gather(0)
        scatter_add(0)
        cnt_scatter(1)
        wait_gather(1)
        scatter_add(1)

        base = tile_base + NFULL * CHUNK
        pltpu.sync_copy(gidx_hbm.at[pl.ds(base, TAIL)], gi_t.at[0])
        pltpu.sync_copy(sidx_hbm.at[pl.ds(base, TAIL)], si_t.at[0])
        pltpu.async_copy(table_hbm.at[gi_t.at[0]],
                         b0_v.at[pl.ds(0, TAIL)], gsem0).wait()
        for j in range(TAIL // 16):
            idx = si_t[0, pl.ds(j * 16, 16)]
            plsc.addupdate_scatter(hist_v, [idx],
                                   jnp.ones((16,), jnp.float32))
        pltpu.sync_copy(b0_v.at[pl.ds(0, TAIL)],
                        acc_sh.at[si_t.at[0]], add=True)

        plsc.subcore_barrier()

        pltpu.sync_copy(hist_v, cnt_hbm.at[c, s])

        @pl.loop(s, NZBLK, step=NS)
        def _(t):
            row0 = t * ZROWS
            pltpu.sync_copy(acc_sh.at[pl.ds(row0, ZROWS)],
                            b1_v.at[pl.ds(0, ZROWS)])
            pltpu.sync_copy(b1_v.at[pl.ds(0, ZROWS)],
                            out_hbm.at[c, pl.ds(row0, ZROWS)])

    zf = jnp.zeros((ZROWS, D), jnp.float32)
    return seg_kernel(table, gidx, sidx, zf)


def _tc_front(x, W1, b1, Wu, bu):
    def body(x_ref, w1_ref, b1_ref, wu_ref, bu_ref, m_ref, u_ref):
        xb = x_ref[...]
        m_ref[...] = jnp.maximum(
            jnp.dot(xb, w1_ref[...], preferred_element_type=jnp.float32)
            + b1_ref[...], 0.0)
        u_ref[...] = (
            jnp.dot(xb, wu_ref[...], preferred_element_type=jnp.float32)
            + bu_ref[...])

    return pl.pallas_call(
        body,
        grid=(N // BLK,),
        in_specs=[
            pl.BlockSpec((BLK, D), lambda i: (i, 0)),
            pl.BlockSpec((D, D), lambda i: (0, 0)),
            pl.BlockSpec((1, D), lambda i: (0, 0)),
            pl.BlockSpec((D, D), lambda i: (0, 0)),
            pl.BlockSpec((1, D), lambda i: (0, 0)),
        ],
        out_specs=[
            pl.BlockSpec((BLK, D), lambda i: (i, 0)),
            pl.BlockSpec((BLK, D), lambda i: (i, 0)),
        ],
        out_shape=[
            jax.ShapeDtypeStruct((N, D), jnp.float32),
            jax.ShapeDtypeStruct((N, D), jnp.float32),
        ],
    )(x, W1, b1.reshape(1, D), Wu, bu.reshape(1, D))


NPAD = 10240
CBLK = 1280


def _tc_cnt(cnts):
    padded = jnp.pad(cnts.reshape(NC * NS, N), ((0, 0), (0, NPAD - N)))

    def body(c_ref, o_ref):
        ones = jnp.ones((NC * NS, 1), jnp.float32)
        o_ref[...] = lax.dot_general(
            c_ref[...], ones, (((0,), (0,)), ((), ())),
            preferred_element_type=jnp.float32)

    out = pl.pallas_call(
        body,
        grid=(NPAD // CBLK,),
        in_specs=[pl.BlockSpec((NC * NS, CBLK), lambda i: (0, i))],
        out_specs=pl.BlockSpec((CBLK, 1), lambda i: (i, 0)),
        out_shape=jax.ShapeDtypeStruct((NPAD, 1), jnp.float32),
    )(padded)
    return out[:N]


def _tc_mid(parts, cnt_col):
    def body(p_ref, c_ref, o_ref):
        o_ref[...] = (p_ref[0] + p_ref[1]) / jnp.maximum(c_ref[...], 1.0)

    return pl.pallas_call(
        body,
        grid=(E // BLK,),
        in_specs=[
            pl.BlockSpec((NC, BLK, D), lambda i: (0, i, 0)),
            pl.BlockSpec((BLK, 1), lambda i: (i, 0)),
        ],
        out_specs=pl.BlockSpec((BLK, D), lambda i: (i, 0)),
        out_shape=jax.ShapeDtypeStruct((E, D), jnp.float32),
    )(parts, cnt_col)


def _tc_back(u, parts, cnt_col):
    def body(u_ref, p_ref, c_ref, o_ref):
        mi = (p_ref[0] + p_ref[1]) / jnp.maximum(c_ref[...], 1.0)
        h = jnp.maximum(u_ref[...] + mi, 0.0)
        mx = jnp.max(h, axis=1, keepdims=True)
        lse = jnp.log(jnp.sum(jnp.exp(h - mx), axis=1, keepdims=True))
        o_ref[...] = h - mx - lse

    return pl.pallas_call(
        body,
        grid=(N // BLK,),
        in_specs=[
            pl.BlockSpec((BLK, D), lambda i: (i, 0)),
            pl.BlockSpec((NC, BLK, D), lambda i: (0, i, 0)),
            pl.BlockSpec((BLK, 1), lambda i: (i, 0)),
        ],
        out_specs=pl.BlockSpec((BLK, D), lambda i: (i, 0)),
        out_shape=jax.ShapeDtypeStruct((N, D), jnp.float32),
    )(u, parts, cnt_col)


def kernel(x, edge_index, W1, b1, Wu, bu):
    v_idx = edge_index[0]
    e_idx = edge_index[1]
    m, u = _tc_front(x, W1, b1, Wu, bu)
    e_parts, e_cnts = _sc_segment_sum(m, v_idx, e_idx)
    e_feat = _tc_mid(e_parts, _tc_cnt(e_cnts))
    v_parts, v_cnts = _sc_segment_sum(e_feat, e_idx, v_idx)
    return _tc_back(u, v_parts, _tc_cnt(v_cnts))


# speedup vs baseline: 1.1847x; 1.0407x over previous
import dataclasses
import functools

import jax
import jax.numpy as jnp
from jax import lax
from jax.experimental import pallas as pl
from jax.experimental.pallas import tpu as pltpu
from jax.experimental.pallas import tpu_sc as plsc

N = 10000
E = 10000
NI = 320000
D = 128
CW = 16
NC = 2
NS = 16
CHUNK = 128
PER_TILE = NI // (NC * NS)
NFULL = (PER_TILE // CHUNK) - (PER_TILE // CHUNK) % 2
TAIL = PER_TILE - NFULL * CHUNK
ZROWS = 80
NZBLK = N // ZROWS
BLK = 1000
NPAD = 10240
CBLK = 1280


def _sc_segment_sum(table, gidx, sidx):
    mesh = plsc.VectorSubcoreMesh(core_axis_name="c", subcore_axis_name="s")

    @functools.partial(
        pl.kernel,
        out_type=[
            jax.ShapeDtypeStruct((NC, N, D), jnp.float32),
            jax.ShapeDtypeStruct((NC, NS, NPAD), jnp.float32),
        ],
        mesh=mesh,
        scratch_types=[
            pltpu.VMEM((2, CHUNK), jnp.int32),
            pltpu.VMEM((2, CHUNK), jnp.int32),
            pltpu.VMEM((1, TAIL), jnp.int32),
            pltpu.VMEM((1, TAIL), jnp.int32),
            pltpu.VMEM((CHUNK, D), jnp.float32),
            pltpu.VMEM((CHUNK, D), jnp.float32),
            pltpu.VMEM((NPAD,), jnp.float32),
            pltpu.VMEM_SHARED((N, D), jnp.float32),
            pltpu.SemaphoreType.DMA,
            pltpu.SemaphoreType.DMA,
            pltpu.SemaphoreType.DMA,
            pltpu.SemaphoreType.DMA,
        ],
        compiler_params=dataclasses.replace(
            pltpu.CompilerParams(), needs_layout_passes=False),
    )
    def seg_kernel(table_hbm, gidx_hbm, sidx_hbm, zf_hbm,
                   out_hbm, cnt_hbm,
                   gi_v, si_v, gi_t, si_t, b0_v, b1_v, hist_v, acc_sh,
                   gsem0, gsem1, isem0, isem1):
        c = lax.axis_index("c")
        s = lax.axis_index("s")
        tile_base = (c * NS + s) * PER_TILE
        bufs = (b0_v, b1_v)
        gsems = (gsem0, gsem1)
        isems = (isem0, isem1)

        @pl.loop(0, NPAD // 16)
        def _(i):
            hist_v[pl.ds(i * 16, 16)] = jnp.zeros((16,), jnp.float32)

        pltpu.sync_copy(zf_hbm, b0_v.at[pl.ds(0, ZROWS)])

        @pl.loop(s, NZBLK, step=NS)
        def _(t):
            pltpu.sync_copy(b0_v.at[pl.ds(0, ZROWS)],
                            acc_sh.at[pl.ds(t * ZROWS, ZROWS)])

        plsc.subcore_barrier()

        def chunk_base(i):
            return tile_base + i * CHUNK

        def start_idx_fetch(i, slot):
            base = chunk_base(i)
            pltpu.async_copy(gidx_hbm.at[pl.ds(base, CHUNK)],
                             gi_v.at[slot], isems[slot])
            pltpu.async_copy(sidx_hbm.at[pl.ds(base, CHUNK)],
                             si_v.at[slot], isems[slot])

        def wait_idx(slot):
            pltpu.make_async_copy(gidx_hbm.at[pl.ds(0, CHUNK)],
                                  gi_v.at[slot], isems[slot]).wait()
            pltpu.make_async_copy(gidx_hbm.at[pl.ds(0, CHUNK)],
                                  si_v.at[slot], isems[slot]).wait()

        def start_gather(slot):
            pltpu.async_copy(table_hbm.at[gi_v.at[slot]],
                             bufs[slot], gsems[slot])

        def wait_gather(slot):
            pltpu.make_async_copy(table_hbm.at[pl.ds(0, CHUNK)],
                                  bufs[slot], gsems[slot]).wait()

        def cnt_scatter(slot):
            for j in range(CHUNK // 16):
                idx = si_v[slot, pl.ds(j * 16, 16)]
                plsc.addupdate_scatter(hist_v, [idx],
                                       jnp.ones((16,), jnp.float32))

        def scatter_add(slot):
            pltpu.sync_copy(bufs[slot], acc_sh.at[si_v.at[slot]], add=True)

        base0 = chunk_base(0)
        pltpu.sync_copy(gidx_hbm.at[pl.ds(base0, CHUNK)], gi_v.at[0])
        pltpu.sync_copy(sidx_hbm.at[pl.ds(base0, CHUNK)], si_v.at[0])
        start_idx_fetch(1, 1)
        start_gather(0)

        @pl.loop(0, NFULL // 2 - 1)
        def _(p):
            i0 = 2 * p
            wait_idx(1)
            start_gather(1)
            cnt_scatter(0)
            wait_gather(0)
            scatter_add(0)
            start_idx_fetch(i0 + 2, 0)
            wait_idx(0)
            start_gather(0)
            cnt_scatter(1)
            wait_gather(1)
            scatter_add(1)
            start_idx_fetch(i0 + 3, 1)

        wait_idx(1)
        start_gather(1)
        cnt_scatter(0)
        wait_gather(0)
        scatter_add(0)
        cnt_scatter(1)
        wait_gather(1)
        scatter_add(1)

        base = tile_base + NFULL * CHUNK
        pltpu.sync_copy(gidx_hbm.at[pl.ds(base, TAIL)], gi_t.at[0])
        pltpu.sync_copy(sidx_hbm.at[pl.ds(base, TAIL)], si_t.at[0])
        pltpu.async_copy(table_hbm.at[gi_t.at[0]],
                         b0_v.at[pl.ds(0, TAIL)], gsem0).wait()
        for j in range(TAIL // 16):
            idx = si_t[0, pl.ds(j * 16, 16)]
            plsc.addupdate_scatter(hist_v, [idx],
                                   jnp.ones((16,), jnp.float32))
        pltpu.sync_copy(b0_v.at[pl.ds(0, TAIL)],
                        acc_sh.at[si_t.at[0]], add=True)

        plsc.subcore_barrier()

        pltpu.sync_copy(hist_v, cnt_hbm.at[c, s])

        @pl.loop(s, NZBLK, step=NS)
        def _(t):
            row0 = t * ZROWS
            pltpu.sync_copy(acc_sh.at[pl.ds(row0, ZROWS)],
                            b1_v.at[pl.ds(0, ZROWS)])
            pltpu.sync_copy(b1_v.at[pl.ds(0, ZROWS)],
                            out_hbm.at[c, pl.ds(row0, ZROWS)])

    zf = jnp.zeros((ZROWS, D), jnp.float32)
    return seg_kernel(table, gidx, sidx, zf)


def _tc_front(x, W1, b1, Wu, bu):
    def body(x_ref, w1_ref, b1_ref, wu_ref, bu_ref, m_ref, u_ref):
        xb = x_ref[...]
        m_ref[...] = jnp.maximum(
            jnp.dot(xb, w1_ref[...], preferred_element_type=jnp.float32)
            + b1_ref[...], 0.0)
        u_ref[...] = (
            jnp.dot(xb, wu_ref[...], preferred_element_type=jnp.float32)
            + bu_ref[...])

    return pl.pallas_call(
        body,
        grid=(N // BLK,),
        in_specs=[
            pl.BlockSpec((BLK, D), lambda i: (i, 0)),
            pl.BlockSpec((D, D), lambda i: (0, 0)),
            pl.BlockSpec((1, D), lambda i: (0, 0)),
            pl.BlockSpec((D, D), lambda i: (0, 0)),
            pl.BlockSpec((1, D), lambda i: (0, 0)),
        ],
        out_specs=[
            pl.BlockSpec((BLK, D), lambda i: (i, 0)),
            pl.BlockSpec((BLK, D), lambda i: (i, 0)),
        ],
        out_shape=[
            jax.ShapeDtypeStruct((N, D), jnp.float32),
            jax.ShapeDtypeStruct((N, D), jnp.float32),
        ],
    )(x, W1, b1.reshape(1, D), Wu, bu.reshape(1, D))


def _tc_cnt(cnts):
    flat = cnts.reshape(NC * NS, NPAD)

    def body(c_ref, o_ref):
        ones = jnp.ones((NC * NS, 1), jnp.float32)
        col = lax.dot_general(c_ref[...], ones, (((0,), (0,)), ((), ())),
                              preferred_element_type=jnp.float32)
        o_ref[...] = jnp.broadcast_to(col, (CBLK, D))

    return pl.pallas_call(
        body,
        grid=(NPAD // CBLK,),
        in_specs=[pl.BlockSpec((NC * NS, CBLK), lambda i: (0, i))],
        out_specs=pl.BlockSpec((CBLK, D), lambda i: (i, 0)),
        out_shape=jax.ShapeDtypeStruct((NPAD, D), jnp.float32),
    )(flat)


def _tc_mid(parts, cnt):
    def body(p_ref, c_ref, o_ref):
        o_ref[...] = (p_ref[0] + p_ref[1]) / jnp.maximum(c_ref[...], 1.0)

    return pl.pallas_call(
        body,
        grid=(E // BLK,),
        in_specs=[
            pl.BlockSpec((NC, BLK, D), lambda i: (0, i, 0)),
            pl.BlockSpec((BLK, D), lambda i: (i, 0)),
        ],
        out_specs=pl.BlockSpec((BLK, D), lambda i: (i, 0)),
        out_shape=jax.ShapeDtypeStruct((E, D), jnp.float32),
    )(parts, cnt)


def _tc_back(u, parts, cnt):
    def body(u_ref, p_ref, c_ref, o_ref):
        mi = (p_ref[0] + p_ref[1]) / jnp.maximum(c_ref[...], 1.0)
        h = jnp.maximum(u_ref[...] + mi, 0.0)
        mx = jnp.max(h, axis=1, keepdims=True)
        lse = jnp.log(jnp.sum(jnp.exp(h - mx), axis=1, keepdims=True))
        o_ref[...] = h - mx - lse

    return pl.pallas_call(
        body,
        grid=(N // BLK,),
        in_specs=[
            pl.BlockSpec((BLK, D), lambda i: (i, 0)),
            pl.BlockSpec((NC, BLK, D), lambda i: (0, i, 0)),
            pl.BlockSpec((BLK, D), lambda i: (i, 0)),
        ],
        out_specs=pl.BlockSpec((BLK, D), lambda i: (i, 0)),
        out_shape=jax.ShapeDtypeStruct((N, D), jnp.float32),
    )(u, parts, cnt)


def kernel(x, edge_index, W1, b1, Wu, bu):
    v_idx = edge_index[0]
    e_idx = edge_index[1]
    m, u = _tc_front(x, W1, b1, Wu, bu)
    e_parts, e_cnts = _sc_segment_sum(m, v_idx, e_idx)
    e_feat = _tc_mid(e_parts, _tc_cnt(e_cnts))
    v_parts, v_cnts = _sc_segment_sum(e_feat, e_idx, v_idx)
    return _tc_back(u, v_parts, _tc_cnt(v_cnts))
